# baseline (speedup 1.0000x reference)
import functools

import jax
import jax.numpy as jnp
from jax import lax
from jax.experimental import pallas as pl
from jax.experimental.pallas import tpu as pltpu
from jax.experimental.pallas import tpu_sc as plsc

VOCAB = 100000
D = 64
BATCH = 1024
CAP = 50
NC, NS = 2, 16
NW = NC * NS
DPW = D // NW


@functools.partial(
    pl.kernel,
    mesh=plsc.VectorSubcoreMesh(core_axis_name="c", subcore_axis_name="s"),
    out_type=jax.ShapeDtypeStruct((8 * CAP, 8, 8, 128), jnp.float32),
    scratch_types=[
        pltpu.VMEM((VOCAB,), jnp.float32),
        pltpu.VMEM((BATCH,), jnp.int32),
        pltpu.VMEM((8, 128), jnp.float32),
        pltpu.SemaphoreType.DMA,
    ],
    compiler_params=pltpu.CompilerParams(needs_layout_passes=False),
)
def _gather_t_kernel(tab_hbm, idx_hbm, out_hbm, row_v, idx_v, blk_v, sem):
    wid = lax.axis_index("s") * NC + lax.axis_index("c")

    def per_dim(h, _):
        d = wid * DPW + h
        g = d // 8
        r = d % 8
        pltpu.sync_copy(tab_hbm.at[g, r], row_v)

        def per_t(t, _):
            pltpu.sync_copy(idx_hbm.at[t], idx_v)
            for k in range(BATCH // 16):
                iv = idx_v[pl.ds(k * 16, 16)]
                vals = plsc.load_gather(row_v, [iv])
                blk_v[k // 8, pl.ds((k * 16) % 128, 16)] = vals
            pltpu.sync_copy(blk_v, out_hbm.at[8 * t + g, :, r])
            return ()

        lax.fori_loop(0, CAP, per_t, (), unroll=False)
        return ()

    lax.fori_loop(0, DPW, per_dim, (), unroll=False)


def kernel(encoder_out, encoded_captions, caption_lengths, embedding_weight):
    tab3 = embedding_weight.T.reshape(8, 8, VOCAB)
    idx_t = encoded_captions.T
    out4 = _gather_t_kernel(tab3, idx_t)
    out = (
        out4.transpose(0, 2, 1, 3)
        .reshape(8 * CAP * 8, 8 * 128)
        .reshape(CAP, D, BATCH)
        .transpose(2, 0, 1)
    )
    return out


# speedup vs baseline: 1.3545x; 1.3545x over previous
import functools

import jax
import jax.numpy as jnp
from jax import lax
from jax.experimental import pallas as pl
from jax.experimental.pallas import tpu as pltpu
from jax.experimental.pallas import tpu_sc as plsc

VOCAB = 100000
D = 64
BATCH = 1024
CAP = 50
NC, NS = 2, 16
NW = NC * NS
DPW = D // NW


@functools.partial(
    pl.kernel,
    mesh=plsc.VectorSubcoreMesh(core_axis_name="c", subcore_axis_name="s"),
    out_type=jax.ShapeDtypeStruct((8 * CAP, 8, 8, 128), jnp.float32),
    scratch_types=[
        pltpu.VMEM((VOCAB,), jnp.float32),
        pltpu.VMEM((2, BATCH), jnp.int32),
        pltpu.VMEM((2, 8, 128), jnp.float32),
        pltpu.SemaphoreType.DMA((2,)),
        pltpu.SemaphoreType.DMA((2,)),
    ],
    compiler_params=pltpu.CompilerParams(needs_layout_passes=False),
)
def _gather_t_kernel(tab_hbm, idx_hbm, out_hbm, row_v, idx_v, blk_v,
                     sem_idx, sem_out):
    wid = lax.axis_index("s") * NC + lax.axis_index("c")

    def per_dim(h, _):
        d = wid * DPW + jnp.bitwise_xor(h, jnp.bitwise_and(wid, 1))
        g = d // 8
        r = d % 8
        pltpu.sync_copy(tab_hbm.at[g, r], row_v)
        pltpu.async_copy(idx_hbm.at[0], idx_v.at[0], sem_idx.at[0])
        pltpu.async_copy(idx_hbm.at[1], idx_v.at[1], sem_idx.at[1])

        def per_t(t, _):
            p = lax.rem(t, 2)
            pltpu.make_async_copy(
                idx_hbm.at[0], idx_v.at[p], sem_idx.at[p]).wait()

            @pl.when(t >= 2)
            def _():
                pltpu.make_async_copy(
                    blk_v.at[p], out_hbm.at[0, :, 0], sem_out.at[p]).wait()

            for k in range(BATCH // 16):
                iv = idx_v[p, pl.ds(k * 16, 16)]
                vals = plsc.load_gather(row_v, [iv])
                blk_v[p, k // 8, pl.ds((k * 16) % 128, 16)] = vals

            pltpu.async_copy(
                blk_v.at[p], out_hbm.at[8 * t + g, :, r], sem_out.at[p])

            @pl.when(t + 2 < CAP)
            def _():
                pltpu.async_copy(idx_hbm.at[t + 2], idx_v.at[p], sem_idx.at[p])

            return ()

        lax.fori_loop(0, CAP, per_t, (), unroll=False)
        pltpu.make_async_copy(
            blk_v.at[0], out_hbm.at[0, :, 0], sem_out.at[0]).wait()
        pltpu.make_async_copy(
            blk_v.at[1], out_hbm.at[0, :, 0], sem_out.at[1]).wait()
        return ()

    lax.fori_loop(0, DPW, per_dim, (), unroll=False)


def kernel(encoder_out, encoded_captions, caption_lengths, embedding_weight):
    tab3 = embedding_weight.T.reshape(8, 8, VOCAB)
    idx_t = encoded_captions.T
    out4 = _gather_t_kernel(tab3, idx_t)
    out = (
        out4.transpose(0, 2, 1, 3)
        .reshape(8 * CAP * 8, 8 * 128)
        .reshape(CAP, D, BATCH)
        .transpose(2, 0, 1)
    )
    return out


# speedup vs baseline: 1.7442x; 1.2877x over previous
import functools

import jax
import jax.numpy as jnp
from jax import lax
from jax.experimental import pallas as pl
from jax.experimental.pallas import tpu as pltpu
from jax.experimental.pallas import tpu_sc as plsc

VOCAB = 100000
D = 64
BATCH = 1024
CAP = 50
NC, NS = 2, 16
NW = NC * NS
DPW = D // NW


@functools.partial(
    pl.kernel,
    mesh=plsc.VectorSubcoreMesh(core_axis_name="c", subcore_axis_name="s"),
    out_type=jax.ShapeDtypeStruct((8 * CAP, 8, 8, 128), jnp.float32),
    scratch_types=[
        pltpu.VMEM((VOCAB,), jnp.float32),
        pltpu.VMEM((2, BATCH), jnp.int32),
        pltpu.VMEM((2, 8, 128), jnp.float32),
        pltpu.SemaphoreType.DMA((2,)),
        pltpu.SemaphoreType.DMA((2,)),
    ],
    compiler_params=pltpu.CompilerParams(needs_layout_passes=False),
)
def _gather_t_kernel(tab_hbm, idx_hbm, out_hbm, row_v, idx_v, blk_v,
                     sem_idx, sem_out):
    wid = lax.axis_index("s") * NC + lax.axis_index("c")

    def per_dim(h, _):
        d = wid * DPW + jnp.bitwise_xor(h, jnp.bitwise_and(wid, 1))
        g = d // 8
        r = d % 8
        pltpu.sync_copy(tab_hbm.at[g, r], row_v)
        pltpu.async_copy(idx_hbm.at[0], idx_v.at[0], sem_idx.at[0])
        pltpu.async_copy(idx_hbm.at[1], idx_v.at[1], sem_idx.at[1])

        def per_t(t, _):
            p = lax.rem(t, 2)
            pltpu.make_async_copy(
                idx_hbm.at[0], idx_v.at[p], sem_idx.at[p]).wait()

            @pl.when(t >= 2)
            def _():
                pltpu.make_async_copy(
                    blk_v.at[p], out_hbm.at[0, :, 0], sem_out.at[p]).wait()

            @plsc.parallel_loop(0, BATCH // 16, unroll=8)
            def _(k):
                iv = idx_v[p, pl.ds(k * 16, 16)]
                vals = plsc.load_gather(row_v, [iv])
                blk_v[p, k // 8, pl.ds((k * 16) % 128, 16)] = vals

            pltpu.async_copy(
                blk_v.at[p], out_hbm.at[8 * t + g, :, r], sem_out.at[p])

            @pl.when(t + 2 < CAP)
            def _():
                pltpu.async_copy(idx_hbm.at[t + 2], idx_v.at[p], sem_idx.at[p])

            return ()

        lax.fori_loop(0, CAP, per_t, (), unroll=False)
        pltpu.make_async_copy(
            blk_v.at[0], out_hbm.at[0, :, 0], sem_out.at[0]).wait()
        pltpu.make_async_copy(
            blk_v.at[1], out_hbm.at[0, :, 0], sem_out.at[1]).wait()
        return ()

    lax.fori_loop(0, DPW, per_dim, (), unroll=False)


def kernel(encoder_out, encoded_captions, caption_lengths, embedding_weight):
    tab3 = embedding_weight.T.reshape(8, 8, VOCAB)
    idx_t = encoded_captions.T
    out4 = _gather_t_kernel(tab3, idx_t)
    out = (
        out4.transpose(0, 2, 1, 3)
        .reshape(8 * CAP * 8, 8 * 128)
        .reshape(CAP, D, BATCH)
        .transpose(2, 0, 1)
    )
    return out


# speedup vs baseline: 1.7461x; 1.0011x over previous
import functools

import jax
import jax.numpy as jnp
from jax import lax
from jax.experimental import pallas as pl
from jax.experimental.pallas import tpu as pltpu
from jax.experimental.pallas import tpu_sc as plsc

VOCAB = 100000
D = 64
BATCH = 1024
CAP = 50
NC, NS = 2, 16
NW = NC * NS
DPW = D // NW


@functools.partial(
    pl.kernel,
    mesh=plsc.VectorSubcoreMesh(core_axis_name="c", subcore_axis_name="s"),
    out_type=jax.ShapeDtypeStruct((8 * CAP, 8, 8, 128), jnp.float32),
    scratch_types=[
        pltpu.VMEM((VOCAB,), jnp.float32),
        pltpu.VMEM((2, BATCH), jnp.int32),
        pltpu.VMEM((2, 8, 128), jnp.float32),
        pltpu.SemaphoreType.DMA((2,)),
        pltpu.SemaphoreType.DMA((2,)),
    ],
    compiler_params=pltpu.CompilerParams(needs_layout_passes=False),
)
def _gather_t_kernel(tab_hbm, idx_hbm, out_hbm, row_v, idx_v, blk_v,
                     sem_idx, sem_out):
    wid = lax.axis_index("s") * NC + lax.axis_index("c")

    def per_dim(h, _):
        d = wid * DPW + jnp.bitwise_xor(h, jnp.bitwise_and(wid, 1))
        g = d // 8
        r = d % 8
        pltpu.sync_copy(tab_hbm.at[g, r], row_v)
        pltpu.async_copy(idx_hbm.at[0], idx_v.at[0], sem_idx.at[0])
        pltpu.async_copy(idx_hbm.at[1], idx_v.at[1], sem_idx.at[1])

        def per_t(t, _):
            p = lax.rem(t, 2)
            pltpu.make_async_copy(
                idx_hbm.at[0], idx_v.at[p], sem_idx.at[p]).wait()

            @pl.when(t >= 2)
            def _():
                pltpu.make_async_copy(
                    blk_v.at[p], out_hbm.at[0, :, 0], sem_out.at[p]).wait()

            @plsc.parallel_loop(0, BATCH // 16, unroll=16)
            def _(k):
                iv = idx_v[p, pl.ds(k * 16, 16)]
                vals = plsc.load_gather(row_v, [iv])
                blk_v[p, k // 8, pl.ds((k * 16) % 128, 16)] = vals

            pltpu.async_copy(
                blk_v.at[p], out_hbm.at[8 * t + g, :, r], sem_out.at[p])

            @pl.when(t + 2 < CAP)
            def _():
                pltpu.async_copy(idx_hbm.at[t + 2], idx_v.at[p], sem_idx.at[p])

            return ()

        lax.fori_loop(0, CAP, per_t, (), unroll=False)
        pltpu.make_async_copy(
            blk_v.at[0], out_hbm.at[0, :, 0], sem_out.at[0]).wait()
        pltpu.make_async_copy(
            blk_v.at[1], out_hbm.at[0, :, 0], sem_out.at[1]).wait()
        return ()

    lax.fori_loop(0, DPW, per_dim, (), unroll=False)


def kernel(encoder_out, encoded_captions, caption_lengths, embedding_weight):
    tab3 = embedding_weight.T.reshape(8, 8, VOCAB)
    idx_t = encoded_captions.T
    out4 = _gather_t_kernel(tab3, idx_t)
    out = (
        out4.transpose(0, 2, 1, 3)
        .reshape(8 * CAP * 8, 8 * 128)
        .reshape(CAP, D, BATCH)
        .transpose(2, 0, 1)
    )
    return out


# speedup vs baseline: 2.3657x; 1.3549x over previous
import functools

import jax
import jax.numpy as jnp
from jax import lax
from jax.experimental import pallas as pl
from jax.experimental.pallas import tpu as pltpu
from jax.experimental.pallas import tpu_sc as plsc

VOCAB = 100000
D = 64
BATCH = 1024
CAP = 50
NC, NS = 2, 16
NW = NC * NS
DPW = D // NW
TB = 5


@functools.partial(
    pl.kernel,
    mesh=plsc.VectorSubcoreMesh(core_axis_name="c", subcore_axis_name="s"),
    out_type=jax.ShapeDtypeStruct((8 * CAP, 8, 8, 128), jnp.float32),
    scratch_types=[
        pltpu.VMEM((VOCAB,), jnp.float32),
        pltpu.VMEM((2, TB, 1, BATCH), jnp.int32),
        pltpu.VMEM((2, TB, 8, 128), jnp.float32),
        pltpu.SemaphoreType.DMA((2,)),
        pltpu.SemaphoreType.DMA((2,)),
    ],
    compiler_params=pltpu.CompilerParams(needs_layout_passes=False),
)
def _gather_t_kernel(tab_hbm, idx_hbm, out_hbm, row_v, idx_v, blk_v,
                     sem_idx, sem_out):
    wid = lax.axis_index("s") * NC + lax.axis_index("c")
    nit = CAP // TB

    def per_dim(h, _):
        d = wid * DPW + jnp.bitwise_xor(h, jnp.bitwise_and(wid, 1))
        g = d // 8
        r = d % 8
        pltpu.sync_copy(tab_hbm.at[g, r], row_v)
        dz = h * 0
        for j in range(TB):
            pltpu.async_copy(
                idx_hbm.at[pl.ds(dz + j, 1)], idx_v.at[0, j], sem_idx.at[0])
            pltpu.async_copy(
                idx_hbm.at[pl.ds(dz + TB + j, 1)], idx_v.at[1, j], sem_idx.at[1])

        def per_it(i, _):
            p = lax.rem(i, 2)
            t0 = i * TB
            for _j in range(TB):
                pltpu.make_async_copy(
                    idx_hbm.at[pl.ds(0, 1)], idx_v.at[p, 0], sem_idx.at[p]).wait()

            @pl.when(i >= 2)
            def _():
                for _j in range(TB):
                    pltpu.make_async_copy(
                        blk_v.at[p, 0], out_hbm.at[0, :, 0],
                        sem_out.at[p]).wait()

            @plsc.parallel_loop(0, TB * (BATCH // 16), unroll=16)
            def _(k):
                iv = idx_v[p, k // 64, 0, pl.ds((k * 16) % BATCH, 16)]
                vals = plsc.load_gather(row_v, [iv])
                blk_v[p, k // 64, (k % 64) // 8,
                      pl.ds((k * 16) % 128, 16)] = vals

            for j in range(TB):
                pltpu.async_copy(
                    blk_v.at[p, j], out_hbm.at[8 * (t0 + j) + g, :, r],
                    sem_out.at[p])

            @pl.when(i + 2 < nit)
            def _():
                for j in range(TB):
                    pltpu.async_copy(
                        idx_hbm.at[pl.ds(t0 + 2 * TB + j, 1)], idx_v.at[p, j],
                        sem_idx.at[p])

            return ()

        lax.fori_loop(0, nit, per_it, (), unroll=False)
        for p in range(2):
            for _j in range(TB):
                pltpu.make_async_copy(
                    blk_v.at[p, 0], out_hbm.at[0, :, 0], sem_out.at[p]).wait()
        return ()

    lax.fori_loop(0, DPW, per_dim, (), unroll=False)


def kernel(encoder_out, encoded_captions, caption_lengths, embedding_weight):
    tab3 = embedding_weight.T.reshape(8, 8, VOCAB)
    idx_t = encoded_captions.T
    out4 = _gather_t_kernel(tab3, idx_t)
    out = (
        out4.transpose(0, 2, 1, 3)
        .reshape(8 * CAP * 8, 8 * 128)
        .reshape(CAP, D, BATCH)
        .transpose(2, 0, 1)
    )
    return out
